# baseline (speedup 1.0000x reference)
import functools

import jax
import jax.numpy as jnp
from jax import lax
from jax.experimental import pallas as pl
from jax.experimental.pallas import tpu as pltpu
from jax.experimental.pallas import tpu_sc as plsc

N = 10000
NPAD = 10240
F = 256
E = 160000
HALF = F // 2
L = 16
NS = 16
EPT = E // NS
CHUNK = 80
NCHUNK = EPT // CHUNK
RPT = NPAD // NS
BN = 400


def _proj_body(h_ref, qwt_ref, qb_ref, o_ref):
    y = jnp.dot(h_ref[...], qwt_ref[...], preferred_element_type=jnp.float32)
    y = y + qb_ref[...]
    o_ref[...] = jnp.where(y > 0, y, 0.01 * y).astype(jnp.bfloat16)


def _project(h, qwt, qb2):
    return pl.pallas_call(
        _proj_body,
        grid=(2, N // BN),
        in_specs=[
            pl.BlockSpec((BN, F), lambda p, i: (i, 0)),
            pl.BlockSpec((F, HALF), lambda p, i: (0, p)),
            pl.BlockSpec((1, HALF), lambda p, i: (0, p)),
        ],
        out_specs=pl.BlockSpec((BN, HALF), lambda p, i: (p * (N // BN) + i, 0)),
        out_shape=jax.ShapeDtypeStruct((2 * N, HALF), jnp.bfloat16),
    )(h, qwt, qb2)


def _apply_body(h_ref, lo_ref, hi_ref, w_ref, a_ref, b1_ref, b2_ref, wb_ref,
                o_ref):
    w = w_ref[:, 0:1]
    w = jnp.where(w == 0.0, 1.0, w)
    y = jnp.dot(h_ref[...], a_ref[...], preferred_element_type=jnp.float32)
    agg = jnp.dot(lo_ref[...], b1_ref[...], preferred_element_type=jnp.float32)
    agg = agg + jnp.dot(hi_ref[...], b2_ref[...],
                        preferred_element_type=jnp.float32)
    y = y + agg / w + wb_ref[...]
    y = jnp.where(y > 0, y, 0.01 * y)
    nrm = jnp.sqrt(jnp.sum(y * y, axis=1, keepdims=True))
    nrm = jnp.where(nrm == 0.0, 1.0, nrm)
    o_ref[...] = y / nrm


def _apply(h, lo, hi, w16, a, b1, b2, wb2):
    return pl.pallas_call(
        _apply_body,
        grid=(N // BN,),
        in_specs=[
            pl.BlockSpec((BN, F), lambda i: (i, 0)),
            pl.BlockSpec((BN, HALF), lambda i: (i, 0)),
            pl.BlockSpec((BN, HALF), lambda i: (i, 0)),
            pl.BlockSpec((BN, L), lambda i: (i, 0)),
            pl.BlockSpec((F, F), lambda i: (0, 0)),
            pl.BlockSpec((HALF, F), lambda i: (0, 0)),
            pl.BlockSpec((HALF, F), lambda i: (0, 0)),
            pl.BlockSpec((1, F), lambda i: (0, 0)),
        ],
        out_specs=pl.BlockSpec((BN, F), lambda i: (i, 0)),
        out_shape=jax.ShapeDtypeStruct((N, F), jnp.float32),
    )(h, lo, hi, w16, a, b1, b2, wb2)


def _apply_proj_body(h_ref, lo_ref, hi_ref, w_ref, a_ref, b1_ref, b2_ref,
                     wb_ref, qwt_ref, qb_ref, o_ref, t_ref):
    w = w_ref[:, 0:1]
    w = jnp.where(w == 0.0, 1.0, w)
    y = jnp.dot(h_ref[...], a_ref[...], preferred_element_type=jnp.float32)
    agg = jnp.dot(lo_ref[...], b1_ref[...], preferred_element_type=jnp.float32)
    agg = agg + jnp.dot(hi_ref[...], b2_ref[...],
                        preferred_element_type=jnp.float32)
    y = y + agg / w + wb_ref[...]
    y = jnp.where(y > 0, y, 0.01 * y)
    nrm = jnp.sqrt(jnp.sum(y * y, axis=1, keepdims=True))
    nrm = jnp.where(nrm == 0.0, 1.0, nrm)
    y = y / nrm
    o_ref[...] = y
    t = jnp.dot(y, qwt_ref[...], preferred_element_type=jnp.float32)
    t = t + qb_ref[...]
    t = jnp.where(t > 0, t, 0.01 * t).astype(jnp.bfloat16)
    t_ref[0] = t[:, :HALF]
    t_ref[1] = t[:, HALF:]


def _apply_project(h, lo, hi, w16, a, b1, b2, wb2, qwt, qb2):
    return pl.pallas_call(
        _apply_proj_body,
        grid=(N // BN,),
        in_specs=[
            pl.BlockSpec((BN, F), lambda i: (i, 0)),
            pl.BlockSpec((BN, HALF), lambda i: (i, 0)),
            pl.BlockSpec((BN, HALF), lambda i: (i, 0)),
            pl.BlockSpec((BN, L), lambda i: (i, 0)),
            pl.BlockSpec((F, F), lambda i: (0, 0)),
            pl.BlockSpec((HALF, F), lambda i: (0, 0)),
            pl.BlockSpec((HALF, F), lambda i: (0, 0)),
            pl.BlockSpec((1, F), lambda i: (0, 0)),
            pl.BlockSpec((F, F), lambda i: (0, 0)),
            pl.BlockSpec((1, F), lambda i: (0, 0)),
        ],
        out_specs=[
            pl.BlockSpec((BN, F), lambda i: (i, 0)),
            pl.BlockSpec((2, BN, HALF), lambda i: (0, i, 0)),
        ],
        out_shape=[
            jax.ShapeDtypeStruct((N, F), jnp.float32),
            jax.ShapeDtypeStruct((2, N, HALF), jnp.bfloat16),
        ],
    )(h, lo, hi, w16, a, b1, b2, wb2, qwt, qb2)


def _agg_body(table, src2, dst, ppr, out_lo, out_hi, out_w, src_v, dstb,
              gbi, gsc, pprb, wbuf, acc, wsh, gsem0, gsem1, ssem, psem0,
              psem1):
    c = lax.axis_index("c")
    s = lax.axis_index("s")
    rbase = pl.multiple_of(s * RPT, 8)
    ebase = pl.multiple_of(s * EPT, 8)

    pltpu.sync_copy(src2.at[pl.ds(pl.multiple_of(c * E + s * EPT, 8), EPT)],
                    src_v)

    gsem = (gsem0, gsem1)
    psem = (psem0, psem1)

    def _zb(i, carry):
        for k in range(HALF // L):
            gsc[i, pl.ds(L * k, L)] = jnp.zeros((L,), jnp.float32)
        wbuf[i, :] = jnp.zeros((L,), jnp.float32)
        return carry

    lax.fori_loop(0, CHUNK, _zb, 0)
    for t in range(RPT // CHUNK):
        sl = pl.ds(rbase + t * CHUNK, CHUNK)
        pltpu.sync_copy(gsc, acc.at[sl])

        @pl.when(c == 0)
        def _():
            pltpu.sync_copy(wbuf, wsh.at[sl])

    plsc.subcore_barrier()

    def _issue(j, b):
        pltpu.async_copy(table.at[src_v.at[pl.ds(j * CHUNK, CHUNK)]],
                         gbi.at[b], gsem[b])
        bd = pl.multiple_of(ebase + j * CHUNK, 8)
        pltpu.async_copy(dst.at[pl.ds(bd, CHUNK)], dstb.at[b], psem[b])
        pltpu.async_copy(ppr.at[pl.ds(bd, CHUNK)], pprb.at[b], psem[b])

    def _process(j, b):
        pltpu.make_async_copy(table.at[src_v.at[pl.ds(j * CHUNK, CHUNK)]],
                              gbi.at[b], gsem[b]).wait()
        pltpu.make_async_copy(dst.at[pl.ds(0, CHUNK)], dstb.at[b],
                              psem[b]).wait()
        pltpu.make_async_copy(ppr.at[pl.ds(0, CHUNK)], pprb.at[b],
                              psem[b]).wait()

        def _grp(g, rc):
            pv = pprb[b, pl.ds(g * L, L)]
            for r in range(L):
                spl = jnp.broadcast_to(pv[r], (L,))
                row = g * L + r
                for k in range(HALF // (2 * L)):
                    v = gbi[b, row, pl.ds(L * k, L)]
                    ev = jax.lax.bitcast_convert_type(
                        jnp.left_shift(v, 16), jnp.float32)
                    od = jax.lax.bitcast_convert_type(
                        jnp.bitwise_and(v, jnp.int32(-65536)), jnp.float32)
                    gsc[row, pl.ds(L * k, L)] = ev * spl
                    gsc[row, pl.ds(HALF // 2 + L * k, L)] = od * spl
                wbuf[row, :] = spl
            return rc

        lax.fori_loop(0, CHUNK // L, _grp, 0)
        pltpu.async_copy(gsc, acc.at[dstb.at[b]], ssem, add=True).wait()

        @pl.when(c == 0)
        def _():
            pltpu.sync_copy(wbuf, wsh.at[dstb.at[b]], add=True)

    _issue(0, 0)

    def _pair(p, carry):
        j0 = 2 * p
        _issue(j0 + 1, 1)
        _process(j0, 0)
        _issue(j0 + 2, 0)
        _process(j0 + 1, 1)
        return carry

    lax.fori_loop(0, (NCHUNK - 1) // 2, _pair, 0)
    _process(NCHUNK - 1, 0)
    plsc.subcore_barrier()

    @pl.when(c == 0)
    def _():
        pltpu.sync_copy(acc.at[pl.ds(rbase, RPT)], out_lo.at[pl.ds(rbase, RPT)])
        pltpu.sync_copy(wsh.at[pl.ds(rbase, RPT)], out_w.at[pl.ds(rbase, RPT)])

    @pl.when(c == 1)
    def _():
        pltpu.sync_copy(acc.at[pl.ds(rbase, RPT)], out_hi.at[pl.ds(rbase, RPT)])


@functools.partial(jax.jit, static_argnames=())
def _aggregate(table, src2, dst, ppr):
    mesh = plsc.VectorSubcoreMesh(core_axis_name="c", subcore_axis_name="s")
    return pl.kernel(
        _agg_body,
        out_type=[
            jax.ShapeDtypeStruct((NPAD, HALF), jnp.float32),
            jax.ShapeDtypeStruct((NPAD, HALF), jnp.float32),
            jax.ShapeDtypeStruct((NPAD, L), jnp.float32),
        ],
        mesh=mesh,
        scratch_types=[
            pltpu.VMEM((EPT,), jnp.int32),
            pltpu.VMEM((2, CHUNK), jnp.int32),
            pltpu.VMEM((2, CHUNK, HALF // 2), jnp.int32),
            pltpu.VMEM((CHUNK, HALF), jnp.float32),
            pltpu.VMEM((2, CHUNK), jnp.float32),
            pltpu.VMEM((CHUNK, L), jnp.float32),
            pltpu.VMEM_SHARED((NPAD, HALF), jnp.float32),
            pltpu.VMEM_SHARED((NPAD, L), jnp.float32),
        ] + [pltpu.SemaphoreType.DMA] * 5,
        compiler_params=pltpu.CompilerParams(use_tc_tiling_on_sc=False),
    )(table, src2, dst, ppr)


def kernel(x, edge_index, ppr_weight, Q0_w, Q0_b, W0_w, W0_b, Q1_w, Q1_b,
           W1_w, W1_b):
    src = edge_index[0]
    dst = edge_index[1]
    src2 = jnp.concatenate([src, src + N])

    perm = jnp.concatenate([jnp.arange(0, HALF, 2), jnp.arange(1, HALF, 2)])

    def wparts(ww, wb):
        wwt = ww.T
        return (wwt[:F], wwt[F:F + HALF][perm], wwt[F + HALF:][perm],
                wb.reshape(1, F))

    def bits(t):
        return jax.lax.bitcast_convert_type(
            t.reshape(2 * N, HALF // 2, 2), jnp.int32)

    a0, b10, b20, wb0 = wparts(W0_w, W0_b)
    a1, b11, b21, wb1 = wparts(W1_w, W1_b)

    table = _project(x, Q0_w.T, Q0_b.reshape(1, F))
    lo, hi, w16 = _aggregate(bits(table), src2, dst, ppr_weight)
    h1, table3 = _apply_project(x, lo, hi, w16, a0, b10, b20, wb0,
                                Q1_w.T, Q1_b.reshape(1, F))
    lo, hi, w16 = _aggregate(bits(table3), src2, dst, ppr_weight)
    return _apply(h1, lo, hi, w16, a1, b11, b21, wb1)


# speedup vs baseline: 1.9898x; 1.9898x over previous
import functools

import jax
import jax.numpy as jnp
from jax import lax
from jax.experimental import pallas as pl
from jax.experimental.pallas import tpu as pltpu
from jax.experimental.pallas import tpu_sc as plsc

N = 10000
NPAD = 10240
F = 256
E = 160000
HALF = F // 2
L = 16
NS = 16
EPT = E // NS
CHUNK = 80
NCHUNK = EPT // CHUNK
RPT = NPAD // NS
BN = 400


def _proj_body(h_ref, qwt_ref, qb_ref, o_ref):
    y = jnp.dot(h_ref[...], qwt_ref[...], preferred_element_type=jnp.float32)
    y = y + qb_ref[...]
    o_ref[...] = jnp.where(y > 0, y, 0.01 * y)


def _project(h, qwt, qb2):
    return pl.pallas_call(
        _proj_body,
        grid=(2, N // BN),
        in_specs=[
            pl.BlockSpec((BN, F), lambda p, i: (i, 0)),
            pl.BlockSpec((F, HALF), lambda p, i: (0, p)),
            pl.BlockSpec((1, HALF), lambda p, i: (0, p)),
        ],
        out_specs=pl.BlockSpec((BN, HALF), lambda p, i: (p * (N // BN) + i, 0)),
        out_shape=jax.ShapeDtypeStruct((2 * N, HALF), jnp.float32),
    )(h, qwt, qb2)


def _apply_body(h_ref, lo_ref, hi_ref, w_ref, a_ref, b1_ref, b2_ref, wb_ref,
                o_ref):
    w = w_ref[:, 0:1]
    w = jnp.where(w == 0.0, 1.0, w)
    y = jnp.dot(h_ref[...], a_ref[...], preferred_element_type=jnp.float32)
    agg = jnp.dot(lo_ref[...], b1_ref[...], preferred_element_type=jnp.float32)
    agg = agg + jnp.dot(hi_ref[...], b2_ref[...],
                        preferred_element_type=jnp.float32)
    y = y + agg / w + wb_ref[...]
    y = jnp.where(y > 0, y, 0.01 * y)
    nrm = jnp.sqrt(jnp.sum(y * y, axis=1, keepdims=True))
    nrm = jnp.where(nrm == 0.0, 1.0, nrm)
    o_ref[...] = y / nrm


def _apply(h, lo, hi, w16, a, b1, b2, wb2):
    return pl.pallas_call(
        _apply_body,
        grid=(N // BN,),
        in_specs=[
            pl.BlockSpec((BN, F), lambda i: (i, 0)),
            pl.BlockSpec((BN, HALF), lambda i: (i, 0)),
            pl.BlockSpec((BN, HALF), lambda i: (i, 0)),
            pl.BlockSpec((BN, L), lambda i: (i, 0)),
            pl.BlockSpec((F, F), lambda i: (0, 0)),
            pl.BlockSpec((HALF, F), lambda i: (0, 0)),
            pl.BlockSpec((HALF, F), lambda i: (0, 0)),
            pl.BlockSpec((1, F), lambda i: (0, 0)),
        ],
        out_specs=pl.BlockSpec((BN, F), lambda i: (i, 0)),
        out_shape=jax.ShapeDtypeStruct((N, F), jnp.float32),
    )(h, lo, hi, w16, a, b1, b2, wb2)


def _apply_proj_body(h_ref, lo_ref, hi_ref, w_ref, a_ref, b1_ref, b2_ref,
                     wb_ref, qwt_ref, qb_ref, o_ref, t_ref):
    w = w_ref[:, 0:1]
    w = jnp.where(w == 0.0, 1.0, w)
    y = jnp.dot(h_ref[...], a_ref[...], preferred_element_type=jnp.float32)
    agg = jnp.dot(lo_ref[...], b1_ref[...], preferred_element_type=jnp.float32)
    agg = agg + jnp.dot(hi_ref[...], b2_ref[...],
                        preferred_element_type=jnp.float32)
    y = y + agg / w + wb_ref[...]
    y = jnp.where(y > 0, y, 0.01 * y)
    nrm = jnp.sqrt(jnp.sum(y * y, axis=1, keepdims=True))
    nrm = jnp.where(nrm == 0.0, 1.0, nrm)
    y = y / nrm
    o_ref[...] = y
    t = jnp.dot(y, qwt_ref[...], preferred_element_type=jnp.float32)
    t = t + qb_ref[...]
    t = jnp.where(t > 0, t, 0.01 * t)
    t_ref[0] = t[:, :HALF]
    t_ref[1] = t[:, HALF:]


def _apply_project(h, lo, hi, w16, a, b1, b2, wb2, qwt, qb2):
    return pl.pallas_call(
        _apply_proj_body,
        grid=(N // BN,),
        in_specs=[
            pl.BlockSpec((BN, F), lambda i: (i, 0)),
            pl.BlockSpec((BN, HALF), lambda i: (i, 0)),
            pl.BlockSpec((BN, HALF), lambda i: (i, 0)),
            pl.BlockSpec((BN, L), lambda i: (i, 0)),
            pl.BlockSpec((F, F), lambda i: (0, 0)),
            pl.BlockSpec((HALF, F), lambda i: (0, 0)),
            pl.BlockSpec((HALF, F), lambda i: (0, 0)),
            pl.BlockSpec((1, F), lambda i: (0, 0)),
            pl.BlockSpec((F, F), lambda i: (0, 0)),
            pl.BlockSpec((1, F), lambda i: (0, 0)),
        ],
        out_specs=[
            pl.BlockSpec((BN, F), lambda i: (i, 0)),
            pl.BlockSpec((2, BN, HALF), lambda i: (0, i, 0)),
        ],
        out_shape=[
            jax.ShapeDtypeStruct((N, F), jnp.float32),
            jax.ShapeDtypeStruct((2, N, HALF), jnp.float32),
        ],
    )(h, lo, hi, w16, a, b1, b2, wb2, qwt, qb2)


def _agg_body(table, src2, dst, ppr, out_lo, out_hi, out_w, src_v, dstb,
              gbuf, pprb, wbuf, acc, wsh, gsem0, gsem1, ssem, psem0,
              psem1):
    c = lax.axis_index("c")
    s = lax.axis_index("s")
    rbase = pl.multiple_of(s * RPT, 8)
    ebase = pl.multiple_of(s * EPT, 8)

    pltpu.sync_copy(src2.at[pl.ds(pl.multiple_of(c * E + s * EPT, 8), EPT)],
                    src_v)

    gsem = (gsem0, gsem1)
    psem = (psem0, psem1)

    def _zb(i, carry):
        for k in range(HALF // L):
            gbuf[0, i, pl.ds(L * k, L)] = jnp.zeros((L,), jnp.float32)
        wbuf[i, :] = jnp.zeros((L,), jnp.float32)
        return carry

    lax.fori_loop(0, CHUNK, _zb, 0)
    for t in range(RPT // CHUNK):
        sl = pl.ds(rbase + t * CHUNK, CHUNK)
        pltpu.sync_copy(gbuf.at[0], acc.at[sl])

        @pl.when(c == 0)
        def _():
            pltpu.sync_copy(wbuf, wsh.at[sl])

    plsc.subcore_barrier()

    def _issue(j, b):
        pltpu.async_copy(table.at[src_v.at[pl.ds(j * CHUNK, CHUNK)]],
                         gbuf.at[b], gsem[b])
        bd = pl.multiple_of(ebase + j * CHUNK, 8)
        pltpu.async_copy(dst.at[pl.ds(bd, CHUNK)], dstb.at[b], psem[b])
        pltpu.async_copy(ppr.at[pl.ds(bd, CHUNK)], pprb.at[b], psem[b])

    def _process(j, b):
        pltpu.make_async_copy(table.at[src_v.at[pl.ds(j * CHUNK, CHUNK)]],
                              gbuf.at[b], gsem[b]).wait()
        pltpu.make_async_copy(dst.at[pl.ds(0, CHUNK)], dstb.at[b],
                              psem[b]).wait()
        pltpu.make_async_copy(ppr.at[pl.ds(0, CHUNK)], pprb.at[b],
                              psem[b]).wait()

        def _grp(g, rc):
            pv = pprb[b, pl.ds(g * L, L)]
            for r in range(L):
                spl = jnp.broadcast_to(pv[r], (L,))
                row = g * L + r
                for k in range(HALF // L):
                    sl = pl.ds(L * k, L)
                    gbuf[b, row, sl] = gbuf[b, row, sl] * spl
                wbuf[row, :] = spl
            return rc

        lax.fori_loop(0, CHUNK // L, _grp, 0)
        pltpu.async_copy(gbuf.at[b], acc.at[dstb.at[b]], ssem,
                         add=True).wait()

        @pl.when(c == 0)
        def _():
            pltpu.sync_copy(wbuf, wsh.at[dstb.at[b]], add=True)

    _issue(0, 0)

    def _pair(p, carry):
        j0 = 2 * p
        _issue(j0 + 1, 1)
        _process(j0, 0)
        _issue(j0 + 2, 0)
        _process(j0 + 1, 1)
        return carry

    lax.fori_loop(0, (NCHUNK - 1) // 2, _pair, 0)
    _process(NCHUNK - 1, 0)
    plsc.subcore_barrier()

    @pl.when(c == 0)
    def _():
        pltpu.sync_copy(acc.at[pl.ds(rbase, RPT)], out_lo.at[pl.ds(rbase, RPT)])
        pltpu.sync_copy(wsh.at[pl.ds(rbase, RPT)], out_w.at[pl.ds(rbase, RPT)])

    @pl.when(c == 1)
    def _():
        pltpu.sync_copy(acc.at[pl.ds(rbase, RPT)], out_hi.at[pl.ds(rbase, RPT)])


@functools.partial(jax.jit, static_argnames=())
def _aggregate(table, src2, dst, ppr):
    mesh = plsc.VectorSubcoreMesh(core_axis_name="c", subcore_axis_name="s")
    return pl.kernel(
        _agg_body,
        out_type=[
            jax.ShapeDtypeStruct((NPAD, HALF), jnp.float32),
            jax.ShapeDtypeStruct((NPAD, HALF), jnp.float32),
            jax.ShapeDtypeStruct((NPAD, L), jnp.float32),
        ],
        mesh=mesh,
        scratch_types=[
            pltpu.VMEM((EPT,), jnp.int32),
            pltpu.VMEM((2, CHUNK), jnp.int32),
            pltpu.VMEM((2, CHUNK, HALF), jnp.float32),
            pltpu.VMEM((2, CHUNK), jnp.float32),
            pltpu.VMEM((CHUNK, L), jnp.float32),
            pltpu.VMEM_SHARED((NPAD, HALF), jnp.float32),
            pltpu.VMEM_SHARED((NPAD, L), jnp.float32),
        ] + [pltpu.SemaphoreType.DMA] * 5,
        compiler_params=pltpu.CompilerParams(use_tc_tiling_on_sc=False),
    )(table, src2, dst, ppr)


def kernel(x, edge_index, ppr_weight, Q0_w, Q0_b, W0_w, W0_b, Q1_w, Q1_b,
           W1_w, W1_b):
    src = edge_index[0]
    dst = edge_index[1]
    src2 = jnp.concatenate([src, src + N])

    def wparts(ww, wb):
        wwt = ww.T
        return wwt[:F], wwt[F:F + HALF], wwt[F + HALF:], wb.reshape(1, F)

    a0, b10, b20, wb0 = wparts(W0_w, W0_b)
    a1, b11, b21, wb1 = wparts(W1_w, W1_b)

    table = _project(x, Q0_w.T, Q0_b.reshape(1, F))
    lo, hi, w16 = _aggregate(table, src2, dst, ppr_weight)
    h1, table3 = _apply_project(x, lo, hi, w16, a0, b10, b20, wb0,
                                Q1_w.T, Q1_b.reshape(1, F))
    lo, hi, w16 = _aggregate(table3.reshape(2 * N, HALF), src2, dst,
                             ppr_weight)
    return _apply(h1, lo, hi, w16, a1, b11, b21, wb1)
